# baseline (speedup 1.0000x reference)
import functools

import jax
import jax.numpy as jnp
from jax import lax
from jax.experimental import pallas as pl
from jax.experimental.pallas import tpu as pltpu
from jax.experimental.pallas import tpu_sc as plsc

N = 10000
E = 160000
NPAD = 10240
EPAD = 163840
ROWS_E = EPAD // 128
NSUB = 16
NCORE = 2
ROWS_SUB_N = NPAD // NSUB
WC = 128
F32 = jnp.float32

LAYER_NC = [4, 4, 3, 2, 1]
DIMS_PAD = [512, 512, 384, 256, 128]


def _vector_mesh():
    return plsc.VectorSubcoreMesh(core_axis_name="c", subcore_axis_name="s",
                                  num_cores=NCORE, num_subcores=NSUB)


def _sc_deg(dst_m, ones, zeros):
    rows_half = ROWS_E // NCORE
    rows_sub = rows_half // NSUB

    @functools.partial(
        pl.kernel,
        out_type=jax.ShapeDtypeStruct((NCORE, NPAD, 128), F32),
        mesh=_vector_mesh(),
        scratch_types=[
            pltpu.VMEM((rows_sub, 128), jnp.int32),
            pltpu.VMEM((128, 128), F32),
            pltpu.VMEM_SHARED((NPAD, 128), F32),
        ],
    )
    def k(dst_hbm, ones_hbm, zeros_hbm, out_hbm, didx, ones_v, table):
        core = lax.axis_index("c")
        sub = lax.axis_index("s")
        pltpu.sync_copy(
            dst_hbm.at[pl.ds(core * rows_half + sub * rows_sub, rows_sub)], didx)
        pltpu.sync_copy(ones_hbm, ones_v)
        nbase = sub * ROWS_SUB_N
        pltpu.sync_copy(zeros_hbm.at[pl.ds(nbase, ROWS_SUB_N)],
                        table.at[pl.ds(nbase, ROWS_SUB_N)])
        plsc.subcore_barrier()

        @pl.loop(0, rows_sub)
        def _(t):
            pltpu.sync_copy(ones_v, table.at[didx.at[t]], add=True)

        plsc.subcore_barrier()
        pltpu.sync_copy(table.at[pl.ds(nbase, ROWS_SUB_N)],
                        out_hbm.at[core].at[pl.ds(nbase, ROWS_SUB_N)])

    return k(dst_m, ones, zeros)


def _make_sc_edge(nc):
    rows_half = ROWS_E // NCORE
    rows_sub = rows_half // NSUB

    @functools.partial(
        pl.kernel,
        out_type=jax.ShapeDtypeStruct((NCORE, nc, NPAD, WC), F32),
        mesh=_vector_mesh(),
        scratch_types=[
            pltpu.VMEM((rows_sub, 128), jnp.int32),
            pltpu.VMEM((rows_sub, 128), jnp.int32),
            pltpu.VMEM((128, WC), F32),
            pltpu.VMEM((128, WC), F32),
            pltpu.VMEM_SHARED((NPAD, WC), F32),
            pltpu.SemaphoreType.DMA,
            pltpu.SemaphoreType.DMA,
            pltpu.SemaphoreType.DMA,
            pltpu.SemaphoreType.DMA,
        ],
    )
    def k(y_hbm, src_hbm, dst_hbm, zeros_hbm, out_hbm, sidx, didx, rows0,
          rows1, table, gsem0, gsem1, ssem0, ssem1):
        core = lax.axis_index("c")
        sub = lax.axis_index("s")
        ibase = core * rows_half + sub * rows_sub
        pltpu.sync_copy(src_hbm.at[pl.ds(ibase, rows_sub)], sidx)
        pltpu.sync_copy(dst_hbm.at[pl.ds(ibase, rows_sub)], didx)
        nbase = sub * ROWS_SUB_N
        for c in range(nc):
            @pl.when(core == 0)
            def _():
                pltpu.sync_copy(y_hbm.at[c].at[pl.ds(nbase, ROWS_SUB_N)],
                                table.at[pl.ds(nbase, ROWS_SUB_N)])

            @pl.when(core == 1)
            def _():
                pltpu.sync_copy(zeros_hbm.at[pl.ds(nbase, ROWS_SUB_N)],
                                table.at[pl.ds(nbase, ROWS_SUB_N)])

            plsc.subcore_barrier()

            def _gather(t, buf, sem):
                pltpu.async_copy(y_hbm.at[c].at[sidx.at[t]], buf, sem)

            def _gather_wait(t, buf, sem):
                pltpu.make_async_copy(y_hbm.at[c].at[sidx.at[t]], buf,
                                      sem).wait()

            def _scatter(t, buf, sem):
                pltpu.async_copy(buf, table.at[didx.at[t]], sem, add=True)

            def _scatter_wait(t, buf, sem):
                pltpu.make_async_copy(buf, table.at[didx.at[t]], sem).wait()

            _gather(0, rows0, gsem0)
            _gather_wait(0, rows0, gsem0)
            _scatter(0, rows0, ssem0)
            _gather(1, rows1, gsem1)

            @pl.loop(1, rows_sub - 1, step=2)
            def _(t):
                _scatter_wait(t - 1, rows0, ssem0)
                _gather(t + 1, rows0, gsem0)
                _gather_wait(t, rows1, gsem1)
                _scatter(t, rows1, ssem1)
                _gather_wait(t + 1, rows0, gsem0)
                _scatter(t + 1, rows0, ssem0)
                _scatter_wait(t, rows1, ssem1)
                _gather(t + 2, rows1, gsem1)

            _gather_wait(rows_sub - 1, rows1, gsem1)
            _scatter(rows_sub - 1, rows1, ssem1)
            _scatter_wait(rows_sub - 2, rows0, ssem0)
            _scatter_wait(rows_sub - 1, rows1, ssem1)

            plsc.subcore_barrier()
            pltpu.sync_copy(table.at[pl.ds(nbase, ROWS_SUB_N)],
                            out_hbm.at[core].at[c].at[pl.ds(nbase, ROWS_SUB_N)])

    return k


_NB = 256
_G = NPAD // _NB


def _tc_matmul1(x_p, w1_p):
    def body(x_ref, w_ref, xw_ref):
        xw = jnp.dot(x_ref[...], w_ref[...], preferred_element_type=F32)
        for c in range(4):
            xw_ref[c] = xw[:, c * WC:(c + 1) * WC]

    return pl.pallas_call(
        body,
        grid=(_G,),
        in_specs=[
            pl.BlockSpec((_NB, 32), lambda i: (i, 0)),
            pl.BlockSpec((32, 512), lambda i: (0, 0)),
        ],
        out_specs=pl.BlockSpec((4, _NB, WC), lambda i: (0, i, 0)),
        out_shape=jax.ShapeDtypeStruct((4, NPAD, WC), F32),
    )(x_p, w1_p)


def _tc_dinv_scale(xw, deg):
    def body(xw_ref, deg_ref, y_ref, dinv_ref):
        d = deg_ref[0, :, 0:1] + deg_ref[1, :, 0:1] + 1.0
        dinv = lax.rsqrt(d)
        dinv_ref[...] = dinv
        for c in range(4):
            y_ref[c] = xw_ref[c] * dinv

    return pl.pallas_call(
        body,
        grid=(_G,),
        in_specs=[
            pl.BlockSpec((4, _NB, WC), lambda i: (0, i, 0)),
            pl.BlockSpec((2, _NB, 128), lambda i: (0, i, 0)),
        ],
        out_specs=[
            pl.BlockSpec((4, _NB, WC), lambda i: (0, i, 0)),
            pl.BlockSpec((_NB, 1), lambda i: (i, 0)),
        ],
        out_shape=[
            jax.ShapeDtypeStruct((4, NPAD, WC), F32),
            jax.ShapeDtypeStruct((NPAD, 1), F32),
        ],
    )(xw, deg)


def _tc_mid(s, dinv, b_r, w_r, nc_out):
    _, nc_in, _, _ = s.shape
    d_out = nc_out * WC

    def body(s_ref, dinv_ref, b_ref, w_ref, o_ref):
        dinv = dinv_ref[...]
        acc = jnp.zeros((_NB, d_out), F32)
        for c in range(nc_in):
            h = jnp.maximum((s_ref[0, c] + s_ref[1, c]) * dinv + b_ref[c], 0.0)
            acc = acc + jnp.dot(h, w_ref[c], preferred_element_type=F32)
        y = acc * dinv
        for c2 in range(nc_out):
            o_ref[c2] = y[:, c2 * WC:(c2 + 1) * WC]

    return pl.pallas_call(
        body,
        grid=(_G,),
        in_specs=[
            pl.BlockSpec((2, nc_in, _NB, WC), lambda i: (0, 0, i, 0)),
            pl.BlockSpec((_NB, 1), lambda i: (i, 0)),
            pl.BlockSpec((nc_in, 1, WC), lambda i: (0, 0, 0)),
            pl.BlockSpec((nc_in, WC, d_out), lambda i: (0, 0, 0)),
        ],
        out_specs=pl.BlockSpec((nc_out, _NB, WC), lambda i: (0, i, 0)),
        out_shape=jax.ShapeDtypeStruct((nc_out, NPAD, WC), F32),
    )(s, dinv, b_r, w_r)


def _tc_final(s5, dinv, b_r, batch_p, wlin_p, blin_p):
    def body(s_ref, dinv_ref, b_ref, batch_ref, w_ref, blin_ref, o_ref, pooled):
        i = pl.program_id(0)

        @pl.when(i == 0)
        def _():
            pooled[...] = jnp.full((32, WC), -jnp.inf, F32)

        dinv = dinv_ref[...]
        h = jnp.maximum((s_ref[0, 0] + s_ref[1, 0]) * dinv + b_ref[0], 0.0)
        bvec = batch_ref[...]
        glo = bvec[0, 0]
        ghi = jnp.minimum(bvec[_NB - 1, 0], 31)

        def upd(g, carry):
            m = bvec == g
            vals = jnp.max(jnp.where(m, h, -jnp.inf), axis=0, keepdims=True)
            pooled[pl.ds(g, 1), :] = jnp.maximum(pooled[pl.ds(g, 1), :], vals)
            return carry

        lax.fori_loop(glo, ghi + 1, upd, 0)

        @pl.when(i == _G - 1)
        def _():
            p = pooled[...]
            z = jnp.dot(p, w_ref[...], preferred_element_type=F32)
            o_ref[...] = jax.nn.sigmoid(z + blin_ref[...])

    return pl.pallas_call(
        body,
        grid=(_G,),
        in_specs=[
            pl.BlockSpec((2, 1, _NB, WC), lambda i: (0, 0, i, 0)),
            pl.BlockSpec((_NB, 1), lambda i: (i, 0)),
            pl.BlockSpec((1, 1, WC), lambda i: (0, 0, 0)),
            pl.BlockSpec((_NB, 1), lambda i: (i, 0)),
            pl.BlockSpec((WC, 1), lambda i: (0, 0)),
            pl.BlockSpec((1, 1), lambda i: (0, 0)),
        ],
        out_specs=pl.BlockSpec((32, 1), lambda i: (0, 0)),
        out_shape=jax.ShapeDtypeStruct((32, 1), F32),
        scratch_shapes=[pltpu.VMEM((32, WC), F32)],
    )(s5, dinv, b_r, batch_p, wlin_p, blin_p)


def _pad2(w, r, c):
    return jnp.zeros((r, c), F32).at[:w.shape[0], :w.shape[1]].set(w)


def kernel(x, edge_index, batch, W1, b1, W2, b2, W3, b3, W4, b4, W5, b5,
           W_lin, b_lin):
    sink = (jnp.arange(EPAD - E, dtype=jnp.int32) % (NPAD - N)) + N
    src_m = jnp.concatenate([edge_index[0], sink]).reshape(ROWS_E, 128)
    dst_m = jnp.concatenate([edge_index[1], sink]).reshape(ROWS_E, 128)
    x_p = _pad2(x, NPAD, 32)
    w1_p = _pad2(W1, 32, 512)
    ws = [W2, W3, W4, W5]
    w_rs = []
    for li in range(4):
        w_rs.append(_pad2(ws[li], DIMS_PAD[li], DIMS_PAD[li + 1])
                    .reshape(LAYER_NC[li], WC, DIMS_PAD[li + 1]))
    b_rs = []
    for li, b in enumerate([b1, b2, b3, b4, b5]):
        b_rs.append(jnp.zeros((DIMS_PAD[li],), F32).at[:b.shape[0]].set(b)
                    .reshape(LAYER_NC[li], 1, WC))
    batch_p = jnp.concatenate(
        [batch, jnp.full((NPAD - N,), 32, jnp.int32)]).reshape(NPAD, 1)
    wlin_p = _pad2(W_lin, WC, 1)
    blin_p = b_lin.reshape(1, 1)
    ones = jnp.ones((128, 128), F32)
    zeros = jnp.zeros((NPAD, 128), F32)

    deg = _sc_deg(dst_m, ones, zeros)
    xw = _tc_matmul1(x_p, w1_p)
    y, dinv = _tc_dinv_scale(xw, deg)
    for li in range(5):
        s = _make_sc_edge(LAYER_NC[li])(y, src_m, dst_m, zeros)
        if li < 4:
            y = _tc_mid(s, dinv, b_rs[li], w_rs[li], LAYER_NC[li + 1])
    return _tc_final(s, dinv, b_rs[4], batch_p, wlin_p, blin_p)


# speedup vs baseline: 1.1058x; 1.1058x over previous
import functools

import jax
import jax.numpy as jnp
from jax import lax
from jax.experimental import pallas as pl
from jax.experimental.pallas import tpu as pltpu
from jax.experimental.pallas import tpu_sc as plsc

N = 10000
E = 160000
NPAD = 10240
EPAD = 163840
ROWS_E = EPAD // 128
NSUB = 16
NCORE = 2
ROWS_SUB_N = NPAD // NSUB
WC = 128
F32 = jnp.float32

LAYER_NC = [4, 4, 3, 2, 1]
DIMS_PAD = [512, 512, 384, 256, 128]


def _vector_mesh():
    return plsc.VectorSubcoreMesh(core_axis_name="c", subcore_axis_name="s",
                                  num_cores=NCORE, num_subcores=NSUB)


def _sc_deg(dst_m, ones, zeros):
    rows_half = ROWS_E // NCORE
    rows_sub = rows_half // NSUB

    @functools.partial(
        pl.kernel,
        out_type=jax.ShapeDtypeStruct((NCORE, NPAD, 128), F32),
        mesh=_vector_mesh(),
        scratch_types=[
            pltpu.VMEM((rows_sub, 128), jnp.int32),
            pltpu.VMEM((128, 128), F32),
            pltpu.VMEM_SHARED((NPAD, 128), F32),
        ],
    )
    def k(dst_hbm, ones_hbm, zeros_hbm, out_hbm, didx, ones_v, table):
        core = lax.axis_index("c")
        sub = lax.axis_index("s")
        pltpu.sync_copy(
            dst_hbm.at[pl.ds(core * rows_half + sub * rows_sub, rows_sub)], didx)
        pltpu.sync_copy(ones_hbm, ones_v)
        nbase = sub * ROWS_SUB_N
        pltpu.sync_copy(zeros_hbm.at[pl.ds(nbase, ROWS_SUB_N)],
                        table.at[pl.ds(nbase, ROWS_SUB_N)])
        plsc.subcore_barrier()

        @pl.loop(0, rows_sub)
        def _(t):
            pltpu.sync_copy(ones_v, table.at[didx.at[t]], add=True)

        plsc.subcore_barrier()
        pltpu.sync_copy(table.at[pl.ds(nbase, ROWS_SUB_N)],
                        out_hbm.at[core].at[pl.ds(nbase, ROWS_SUB_N)])

    return k(dst_m, ones, zeros)


def _make_sc_edge(nc, chunk_split):
    rows_sub = 40
    phases = 2 if chunk_split else 1
    ncpc = nc // NCORE
    out_shape = ((nc, NPAD, WC) if chunk_split else (NCORE, nc, NPAD, WC))

    @functools.partial(
        pl.kernel,
        out_type=jax.ShapeDtypeStruct(out_shape, F32),
        mesh=_vector_mesh(),
        scratch_types=[
            pltpu.VMEM((rows_sub, 128), jnp.int32),
            pltpu.VMEM((rows_sub, 128), jnp.int32),
            pltpu.VMEM((128, WC), F32),
            pltpu.VMEM((128, WC), F32),
            pltpu.VMEM_SHARED((NPAD, WC), F32),
            pltpu.SemaphoreType.DMA,
            pltpu.SemaphoreType.DMA,
            pltpu.SemaphoreType.DMA,
            pltpu.SemaphoreType.DMA,
        ],
    )
    def k(y_hbm, src_hbm, dst_hbm, zeros_hbm, out_hbm, sidx, didx, rows0,
          rows1, table, gsem0, gsem1, ssem0, ssem1):
        core = lax.axis_index("c")
        sub = lax.axis_index("s")
        nbase = sub * ROWS_SUB_N

        def edge_pipeline(c):
            def _gather(t, buf, sem):
                pltpu.async_copy(y_hbm.at[c].at[sidx.at[t]], buf, sem)

            def _gather_wait(t, buf, sem):
                pltpu.make_async_copy(y_hbm.at[c].at[sidx.at[t]], buf,
                                      sem).wait()

            def _scatter(t, buf, sem):
                pltpu.async_copy(buf, table.at[didx.at[t]], sem, add=True)

            def _scatter_wait(t, buf, sem):
                pltpu.make_async_copy(buf, table.at[didx.at[t]], sem).wait()

            _gather(0, rows0, gsem0)
            _gather_wait(0, rows0, gsem0)
            _scatter(0, rows0, ssem0)
            _gather(1, rows1, gsem1)

            @pl.loop(1, rows_sub - 1, step=2)
            def _(t):
                _gather_wait(t, rows1, gsem1)
                _scatter(t, rows1, ssem1)
                _scatter_wait(t - 1, rows0, ssem0)
                _gather(t + 1, rows0, gsem0)
                _gather_wait(t + 1, rows0, gsem0)
                _scatter(t + 1, rows0, ssem0)
                _scatter_wait(t, rows1, ssem1)
                _gather(t + 2, rows1, gsem1)

            _gather_wait(rows_sub - 1, rows1, gsem1)
            _scatter(rows_sub - 1, rows1, ssem1)
            _scatter_wait(rows_sub - 2, rows0, ssem0)
            _scatter_wait(rows_sub - 1, rows1, ssem1)

        def load_idx(ibase):
            pltpu.sync_copy(src_hbm.at[pl.ds(ibase, rows_sub)], sidx)
            pltpu.sync_copy(dst_hbm.at[pl.ds(ibase, rows_sub)], didx)

        if chunk_split:
            for j in range(ncpc):
                c = core * ncpc + j
                pltpu.sync_copy(y_hbm.at[c].at[pl.ds(nbase, ROWS_SUB_N)],
                                table.at[pl.ds(nbase, ROWS_SUB_N)])
                plsc.subcore_barrier()
                for p in range(phases):
                    load_idx(sub * (phases * rows_sub) + p * rows_sub)
                    edge_pipeline(c)
                plsc.subcore_barrier()
                pltpu.sync_copy(table.at[pl.ds(nbase, ROWS_SUB_N)],
                                out_hbm.at[c].at[pl.ds(nbase, ROWS_SUB_N)])
        else:
            load_idx(core * (ROWS_E // NCORE) + sub * rows_sub)
            for c in range(nc):
                @pl.when(core == 0)
                def _():
                    pltpu.sync_copy(y_hbm.at[c].at[pl.ds(nbase, ROWS_SUB_N)],
                                    table.at[pl.ds(nbase, ROWS_SUB_N)])

                @pl.when(core == 1)
                def _():
                    pltpu.sync_copy(zeros_hbm.at[pl.ds(nbase, ROWS_SUB_N)],
                                    table.at[pl.ds(nbase, ROWS_SUB_N)])

                plsc.subcore_barrier()
                edge_pipeline(c)
                plsc.subcore_barrier()
                pltpu.sync_copy(
                    table.at[pl.ds(nbase, ROWS_SUB_N)],
                    out_hbm.at[core].at[c].at[pl.ds(nbase, ROWS_SUB_N)])

    return k


_NB = 256
_G = NPAD // _NB


def _tc_matmul1(x_p, w1_p):
    def body(x_ref, w_ref, xw_ref):
        xw = jnp.dot(x_ref[...], w_ref[...], preferred_element_type=F32)
        for c in range(4):
            xw_ref[c] = xw[:, c * WC:(c + 1) * WC]

    return pl.pallas_call(
        body,
        grid=(_G,),
        in_specs=[
            pl.BlockSpec((_NB, 32), lambda i: (i, 0)),
            pl.BlockSpec((32, 512), lambda i: (0, 0)),
        ],
        out_specs=pl.BlockSpec((4, _NB, WC), lambda i: (0, i, 0)),
        out_shape=jax.ShapeDtypeStruct((4, NPAD, WC), F32),
    )(x_p, w1_p)


def _tc_dinv_scale(xw, deg):
    def body(xw_ref, deg_ref, y_ref, dinv_ref):
        d = deg_ref[0, :, 0:1] + deg_ref[1, :, 0:1] + 1.0
        dinv = lax.rsqrt(d)
        dinv_ref[...] = dinv
        for c in range(4):
            y_ref[c] = xw_ref[c] * dinv

    return pl.pallas_call(
        body,
        grid=(_G,),
        in_specs=[
            pl.BlockSpec((4, _NB, WC), lambda i: (0, i, 0)),
            pl.BlockSpec((2, _NB, 128), lambda i: (0, i, 0)),
        ],
        out_specs=[
            pl.BlockSpec((4, _NB, WC), lambda i: (0, i, 0)),
            pl.BlockSpec((_NB, 1), lambda i: (i, 0)),
        ],
        out_shape=[
            jax.ShapeDtypeStruct((4, NPAD, WC), F32),
            jax.ShapeDtypeStruct((NPAD, 1), F32),
        ],
    )(xw, deg)


def _tc_mid(s, dinv, b_r, w_r, nc_out):
    partial = s.ndim == 4
    nc_in = s.shape[1] if partial else s.shape[0]
    d_out = nc_out * WC

    def body(s_ref, dinv_ref, b_ref, w_ref, o_ref):
        dinv = dinv_ref[...]
        acc = jnp.zeros((_NB, d_out), F32)
        for c in range(nc_in):
            sc = (s_ref[0, c] + s_ref[1, c]) if partial else s_ref[c]
            h = jnp.maximum(sc * dinv + b_ref[c], 0.0)
            acc = acc + jnp.dot(h, w_ref[c], preferred_element_type=F32)
        y = acc * dinv
        for c2 in range(nc_out):
            o_ref[c2] = y[:, c2 * WC:(c2 + 1) * WC]

    s_spec = (pl.BlockSpec((2, nc_in, _NB, WC), lambda i: (0, 0, i, 0))
              if partial else
              pl.BlockSpec((nc_in, _NB, WC), lambda i: (0, i, 0)))
    return pl.pallas_call(
        body,
        grid=(_G,),
        in_specs=[
            s_spec,
            pl.BlockSpec((_NB, 1), lambda i: (i, 0)),
            pl.BlockSpec((nc_in, 1, WC), lambda i: (0, 0, 0)),
            pl.BlockSpec((nc_in, WC, d_out), lambda i: (0, 0, 0)),
        ],
        out_specs=pl.BlockSpec((nc_out, _NB, WC), lambda i: (0, i, 0)),
        out_shape=jax.ShapeDtypeStruct((nc_out, NPAD, WC), F32),
    )(s, dinv, b_r, w_r)


def _tc_final(s5, dinv, b_r, batch_p, wlin_p, blin_p):
    def body(s_ref, dinv_ref, b_ref, batch_ref, w_ref, blin_ref, o_ref, pooled):
        i = pl.program_id(0)

        @pl.when(i == 0)
        def _():
            pooled[...] = jnp.full((32, WC), -jnp.inf, F32)

        dinv = dinv_ref[...]
        h = jnp.maximum((s_ref[0, 0] + s_ref[1, 0]) * dinv + b_ref[0], 0.0)
        bvec = batch_ref[...]
        glo = bvec[0, 0]
        ghi = jnp.minimum(bvec[_NB - 1, 0], 31)

        def upd(g, carry):
            m = bvec == g
            vals = jnp.max(jnp.where(m, h, -jnp.inf), axis=0, keepdims=True)
            pooled[pl.ds(g, 1), :] = jnp.maximum(pooled[pl.ds(g, 1), :], vals)
            return carry

        lax.fori_loop(glo, ghi + 1, upd, 0)

        @pl.when(i == _G - 1)
        def _():
            p = pooled[...]
            z = jnp.dot(p, w_ref[...], preferred_element_type=F32)
            o_ref[...] = jax.nn.sigmoid(z + blin_ref[...])

    return pl.pallas_call(
        body,
        grid=(_G,),
        in_specs=[
            pl.BlockSpec((2, 1, _NB, WC), lambda i: (0, 0, i, 0)),
            pl.BlockSpec((_NB, 1), lambda i: (i, 0)),
            pl.BlockSpec((1, 1, WC), lambda i: (0, 0, 0)),
            pl.BlockSpec((_NB, 1), lambda i: (i, 0)),
            pl.BlockSpec((WC, 1), lambda i: (0, 0)),
            pl.BlockSpec((1, 1), lambda i: (0, 0)),
        ],
        out_specs=pl.BlockSpec((32, 1), lambda i: (0, 0)),
        out_shape=jax.ShapeDtypeStruct((32, 1), F32),
        scratch_shapes=[pltpu.VMEM((32, WC), F32)],
    )(s5, dinv, b_r, batch_p, wlin_p, blin_p)


def _pad2(w, r, c):
    return jnp.zeros((r, c), F32).at[:w.shape[0], :w.shape[1]].set(w)


def kernel(x, edge_index, batch, W1, b1, W2, b2, W3, b3, W4, b4, W5, b5,
           W_lin, b_lin):
    sink = (jnp.arange(EPAD - E, dtype=jnp.int32) % (NPAD - N)) + N
    src_m = jnp.concatenate([edge_index[0], sink]).reshape(ROWS_E, 128)
    dst_m = jnp.concatenate([edge_index[1], sink]).reshape(ROWS_E, 128)
    x_p = _pad2(x, NPAD, 32)
    w1_p = _pad2(W1, 32, 512)
    ws = [W2, W3, W4, W5]
    w_rs = []
    for li in range(4):
        w_rs.append(_pad2(ws[li], DIMS_PAD[li], DIMS_PAD[li + 1])
                    .reshape(LAYER_NC[li], WC, DIMS_PAD[li + 1]))
    b_rs = []
    for li, b in enumerate([b1, b2, b3, b4, b5]):
        b_rs.append(jnp.zeros((DIMS_PAD[li],), F32).at[:b.shape[0]].set(b)
                    .reshape(LAYER_NC[li], 1, WC))
    batch_p = jnp.concatenate(
        [batch, jnp.full((NPAD - N,), 32, jnp.int32)]).reshape(NPAD, 1)
    wlin_p = _pad2(W_lin, WC, 1)
    blin_p = b_lin.reshape(1, 1)
    ones = jnp.ones((128, 128), F32)
    zeros = jnp.zeros((NPAD, 128), F32)

    deg = _sc_deg(dst_m, ones, zeros)
    xw = _tc_matmul1(x_p, w1_p)
    y, dinv = _tc_dinv_scale(xw, deg)
    chunk_split = [True, True, False, True, False]
    for li in range(5):
        s = _make_sc_edge(LAYER_NC[li], chunk_split[li])(y, src_m, dst_m,
                                                         zeros)
        if li < 4:
            y = _tc_mid(s, dinv, b_rs[li], w_rs[li], LAYER_NC[li + 1])
    return _tc_final(s, dinv, b_rs[4], batch_p, wlin_p, blin_p)


# speedup vs baseline: 1.1694x; 1.0575x over previous
import functools

import jax
import jax.numpy as jnp
from jax import lax
from jax.experimental import pallas as pl
from jax.experimental.pallas import tpu as pltpu
from jax.experimental.pallas import tpu_sc as plsc

N = 10000
E = 160000
NPAD = 10240
EPAD = 163840
ROWS_E = EPAD // 128
NSUB = 16
NCORE = 2
ROWS_SUB_N = NPAD // NSUB
WC = 128
F32 = jnp.float32

LAYER_NC = [4, 4, 3, 2, 1]
DIMS_PAD = [512, 512, 384, 256, 128]


def _vector_mesh():
    return plsc.VectorSubcoreMesh(core_axis_name="c", subcore_axis_name="s",
                                  num_cores=NCORE, num_subcores=NSUB)


def _sc_deg(dst_m, ones, zeros):
    rows_half = ROWS_E // NCORE
    rows_sub = rows_half // NSUB

    @functools.partial(
        pl.kernel,
        out_type=jax.ShapeDtypeStruct((NCORE, NPAD, 128), F32),
        mesh=_vector_mesh(),
        scratch_types=[
            pltpu.VMEM((rows_sub, 128), jnp.int32),
            pltpu.VMEM((128, 128), F32),
            pltpu.VMEM_SHARED((NPAD, 128), F32),
        ],
    )
    def k(dst_hbm, ones_hbm, zeros_hbm, out_hbm, didx, ones_v, table):
        core = lax.axis_index("c")
        sub = lax.axis_index("s")
        pltpu.sync_copy(
            dst_hbm.at[pl.ds(core * rows_half + sub * rows_sub, rows_sub)], didx)
        pltpu.sync_copy(ones_hbm, ones_v)
        nbase = sub * ROWS_SUB_N
        pltpu.sync_copy(zeros_hbm.at[pl.ds(nbase, ROWS_SUB_N)],
                        table.at[pl.ds(nbase, ROWS_SUB_N)])
        plsc.subcore_barrier()

        @pl.loop(0, rows_sub)
        def _(t):
            pltpu.sync_copy(ones_v, table.at[didx.at[t]], add=True)

        plsc.subcore_barrier()
        pltpu.sync_copy(table.at[pl.ds(nbase, ROWS_SUB_N)],
                        out_hbm.at[core].at[pl.ds(nbase, ROWS_SUB_N)])

    return k(dst_m, ones, zeros)


def _make_sc_edge(nc, chunk_split):
    rows_sub = 40
    phases = 2 if chunk_split else 1
    ncpc = nc // NCORE
    out_shape = ((nc, NPAD, WC) if chunk_split else (NCORE, nc, NPAD, WC))

    @functools.partial(
        pl.kernel,
        out_type=jax.ShapeDtypeStruct(out_shape, F32),
        mesh=_vector_mesh(),
        scratch_types=[
            pltpu.VMEM((rows_sub, 128), jnp.int32),
            pltpu.VMEM((rows_sub, 128), jnp.int32),
            pltpu.VMEM((128, WC), F32),
            pltpu.VMEM((128, WC), F32),
            pltpu.VMEM_SHARED((NPAD, WC), F32),
            pltpu.SemaphoreType.DMA,
            pltpu.SemaphoreType.DMA,
            pltpu.SemaphoreType.DMA,
            pltpu.SemaphoreType.DMA,
        ],
    )
    def k(y_hbm, src_hbm, dst_hbm, zeros_hbm, out_hbm, sidx, didx, rows0,
          rows1, table, gsem0, gsem1, ssem0, ssem1):
        core = lax.axis_index("c")
        sub = lax.axis_index("s")
        nbase = sub * ROWS_SUB_N

        def edge_pipeline(c):
            def _gather(t, buf, sem):
                pltpu.async_copy(y_hbm.at[c].at[sidx.at[t]], buf, sem)

            def _gather_wait(t, buf, sem):
                pltpu.make_async_copy(y_hbm.at[c].at[sidx.at[t]], buf,
                                      sem).wait()

            def _scatter(t, buf, sem):
                pltpu.async_copy(buf, table.at[didx.at[t]], sem, add=True)

            def _scatter_wait(t, buf, sem):
                pltpu.make_async_copy(buf, table.at[didx.at[t]], sem).wait()

            _gather(0, rows0, gsem0)
            _gather_wait(0, rows0, gsem0)
            _scatter(0, rows0, ssem0)
            _gather(1, rows1, gsem1)

            @pl.loop(1, rows_sub - 1, step=2)
            def _(t):
                _gather_wait(t, rows1, gsem1)
                _scatter(t, rows1, ssem1)
                _scatter_wait(t - 1, rows0, ssem0)
                _gather(t + 1, rows0, gsem0)
                _gather_wait(t + 1, rows0, gsem0)
                _scatter(t + 1, rows0, ssem0)
                _scatter_wait(t, rows1, ssem1)
                _gather(t + 2, rows1, gsem1)

            _gather_wait(rows_sub - 1, rows1, gsem1)
            _scatter(rows_sub - 1, rows1, ssem1)
            _scatter_wait(rows_sub - 2, rows0, ssem0)
            _scatter_wait(rows_sub - 1, rows1, ssem1)

        def load_idx(ibase):
            pltpu.sync_copy(src_hbm.at[pl.ds(ibase, rows_sub)], sidx)
            pltpu.sync_copy(dst_hbm.at[pl.ds(ibase, rows_sub)], didx)

        if chunk_split:
            for j in range(ncpc):
                c = core * ncpc + j
                pltpu.sync_copy(y_hbm.at[c].at[pl.ds(nbase, ROWS_SUB_N)],
                                table.at[pl.ds(nbase, ROWS_SUB_N)])
                plsc.subcore_barrier()
                for p in range(phases):
                    load_idx(sub * (phases * rows_sub) + p * rows_sub)
                    edge_pipeline(c)
                plsc.subcore_barrier()
                pltpu.sync_copy(table.at[pl.ds(nbase, ROWS_SUB_N)],
                                out_hbm.at[c].at[pl.ds(nbase, ROWS_SUB_N)])
        else:
            load_idx(core * (ROWS_E // NCORE) + sub * rows_sub)
            for c in range(nc):
                @pl.when(core == 0)
                def _():
                    pltpu.sync_copy(y_hbm.at[c].at[pl.ds(nbase, ROWS_SUB_N)],
                                    table.at[pl.ds(nbase, ROWS_SUB_N)])

                @pl.when(core == 1)
                def _():
                    pltpu.sync_copy(zeros_hbm.at[pl.ds(nbase, ROWS_SUB_N)],
                                    table.at[pl.ds(nbase, ROWS_SUB_N)])

                plsc.subcore_barrier()
                edge_pipeline(c)
                plsc.subcore_barrier()
                pltpu.sync_copy(
                    table.at[pl.ds(nbase, ROWS_SUB_N)],
                    out_hbm.at[core].at[c].at[pl.ds(nbase, ROWS_SUB_N)])

    return k


_NB = 512
_G = NPAD // _NB


def _tc_matmul1(x_p, w1_p):
    def body(x_ref, w_ref, xw_ref):
        xw = jnp.dot(x_ref[...], w_ref[...], preferred_element_type=F32)
        for c in range(4):
            xw_ref[c] = xw[:, c * WC:(c + 1) * WC]

    return pl.pallas_call(
        body,
        grid=(_G,),
        in_specs=[
            pl.BlockSpec((_NB, 32), lambda i: (i, 0)),
            pl.BlockSpec((32, 512), lambda i: (0, 0)),
        ],
        out_specs=pl.BlockSpec((4, _NB, WC), lambda i: (0, i, 0)),
        out_shape=jax.ShapeDtypeStruct((4, NPAD, WC), F32),
    )(x_p, w1_p)


def _tc_dinv_scale(xw, deg):
    def body(xw_ref, deg_ref, y_ref, dinv_ref):
        d = deg_ref[0, :, 0:1] + deg_ref[1, :, 0:1] + 1.0
        dinv = lax.rsqrt(d)
        dinv_ref[...] = dinv
        for c in range(4):
            y_ref[c] = xw_ref[c] * dinv

    return pl.pallas_call(
        body,
        grid=(_G,),
        in_specs=[
            pl.BlockSpec((4, _NB, WC), lambda i: (0, i, 0)),
            pl.BlockSpec((2, _NB, 128), lambda i: (0, i, 0)),
        ],
        out_specs=[
            pl.BlockSpec((4, _NB, WC), lambda i: (0, i, 0)),
            pl.BlockSpec((_NB, 1), lambda i: (i, 0)),
        ],
        out_shape=[
            jax.ShapeDtypeStruct((4, NPAD, WC), F32),
            jax.ShapeDtypeStruct((NPAD, 1), F32),
        ],
    )(xw, deg)


def _tc_mid(s, dinv, b_r, w_r, nc_out):
    partial = s.ndim == 4
    nc_in = s.shape[1] if partial else s.shape[0]
    d_out = nc_out * WC

    def body(s_ref, dinv_ref, b_ref, w_ref, o_ref):
        dinv = dinv_ref[...]
        acc = jnp.zeros((_NB, d_out), F32)
        for c in range(nc_in):
            sc = (s_ref[0, c] + s_ref[1, c]) if partial else s_ref[c]
            h = jnp.maximum(sc * dinv + b_ref[c], 0.0)
            acc = acc + jnp.dot(h, w_ref[c], preferred_element_type=F32)
        y = acc * dinv
        for c2 in range(nc_out):
            o_ref[c2] = y[:, c2 * WC:(c2 + 1) * WC]

    s_spec = (pl.BlockSpec((2, nc_in, _NB, WC), lambda i: (0, 0, i, 0))
              if partial else
              pl.BlockSpec((nc_in, _NB, WC), lambda i: (0, i, 0)))
    return pl.pallas_call(
        body,
        grid=(_G,),
        in_specs=[
            s_spec,
            pl.BlockSpec((_NB, 1), lambda i: (i, 0)),
            pl.BlockSpec((nc_in, 1, WC), lambda i: (0, 0, 0)),
            pl.BlockSpec((nc_in, WC, d_out), lambda i: (0, 0, 0)),
        ],
        out_specs=pl.BlockSpec((nc_out, _NB, WC), lambda i: (0, i, 0)),
        out_shape=jax.ShapeDtypeStruct((nc_out, NPAD, WC), F32),
    )(s, dinv, b_r, w_r)


def _tc_final(s5, dinv, b_r, batch_p, wlin_p, blin_p):
    def body(s_ref, dinv_ref, b_ref, batch_ref, w_ref, blin_ref, o_ref, pooled):
        i = pl.program_id(0)

        @pl.when(i == 0)
        def _():
            pooled[...] = jnp.full((32, WC), -jnp.inf, F32)

        dinv = dinv_ref[...]
        h = jnp.maximum((s_ref[0, 0] + s_ref[1, 0]) * dinv + b_ref[0], 0.0)
        bvec = batch_ref[...]
        glo = bvec[0, 0]
        ghi = jnp.minimum(bvec[_NB - 1, 0], 31)

        def upd(g, carry):
            m = bvec == g
            vals = jnp.max(jnp.where(m, h, -jnp.inf), axis=0, keepdims=True)
            pooled[pl.ds(g, 1), :] = jnp.maximum(pooled[pl.ds(g, 1), :], vals)
            return carry

        lax.fori_loop(glo, ghi + 1, upd, 0)

        @pl.when(i == _G - 1)
        def _():
            p = pooled[...]
            z = jnp.dot(p, w_ref[...], preferred_element_type=F32)
            o_ref[...] = jax.nn.sigmoid(z + blin_ref[...])

    return pl.pallas_call(
        body,
        grid=(_G,),
        in_specs=[
            pl.BlockSpec((2, 1, _NB, WC), lambda i: (0, 0, i, 0)),
            pl.BlockSpec((_NB, 1), lambda i: (i, 0)),
            pl.BlockSpec((1, 1, WC), lambda i: (0, 0, 0)),
            pl.BlockSpec((_NB, 1), lambda i: (i, 0)),
            pl.BlockSpec((WC, 1), lambda i: (0, 0)),
            pl.BlockSpec((1, 1), lambda i: (0, 0)),
        ],
        out_specs=pl.BlockSpec((32, 1), lambda i: (0, 0)),
        out_shape=jax.ShapeDtypeStruct((32, 1), F32),
        scratch_shapes=[pltpu.VMEM((32, WC), F32)],
    )(s5, dinv, b_r, batch_p, wlin_p, blin_p)


def _pad2(w, r, c):
    return jnp.zeros((r, c), F32).at[:w.shape[0], :w.shape[1]].set(w)


def kernel(x, edge_index, batch, W1, b1, W2, b2, W3, b3, W4, b4, W5, b5,
           W_lin, b_lin):
    sink = (jnp.arange(EPAD - E, dtype=jnp.int32) % (NPAD - N)) + N
    src_m = jnp.concatenate([edge_index[0], sink]).reshape(ROWS_E, 128)
    dst_m = jnp.concatenate([edge_index[1], sink]).reshape(ROWS_E, 128)
    x_p = _pad2(x, NPAD, 32)
    w1_p = _pad2(W1, 32, 512)
    ws = [W2, W3, W4, W5]
    w_rs = []
    for li in range(4):
        w_rs.append(_pad2(ws[li], DIMS_PAD[li], DIMS_PAD[li + 1])
                    .reshape(LAYER_NC[li], WC, DIMS_PAD[li + 1]))
    b_rs = []
    for li, b in enumerate([b1, b2, b3, b4, b5]):
        b_rs.append(jnp.zeros((DIMS_PAD[li],), F32).at[:b.shape[0]].set(b)
                    .reshape(LAYER_NC[li], 1, WC))
    batch_p = jnp.concatenate(
        [batch, jnp.full((NPAD - N,), 32, jnp.int32)]).reshape(NPAD, 1)
    wlin_p = _pad2(W_lin, WC, 1)
    blin_p = b_lin.reshape(1, 1)
    ones = jnp.ones((128, 128), F32)
    zeros = jnp.zeros((NPAD, 128), F32)

    deg = _sc_deg(dst_m, ones, zeros)
    xw = _tc_matmul1(x_p, w1_p)
    y, dinv = _tc_dinv_scale(xw, deg)
    chunk_split = [True, True, False, True, False]
    for li in range(5):
        s = _make_sc_edge(LAYER_NC[li], chunk_split[li])(y, src_m, dst_m,
                                                         zeros)
        if li < 4:
            y = _tc_mid(s, dinv, b_rs[li], w_rs[li], LAYER_NC[li + 1])
    return _tc_final(s, dinv, b_rs[4], batch_p, wlin_p, blin_p)


# speedup vs baseline: 1.2021x; 1.0280x over previous
import functools

import jax
import jax.numpy as jnp
from jax import lax
from jax.experimental import pallas as pl
from jax.experimental.pallas import tpu as pltpu
from jax.experimental.pallas import tpu_sc as plsc

N = 10000
E = 160000
NPAD = 10240
EPAD = 163840
ROWS_E = EPAD // 128
NSUB = 16
NCORE = 2
ROWS_SUB_N = NPAD // NSUB
WC = 128
F32 = jnp.float32

LAYER_NC = [4, 4, 3, 2, 1]
DIMS_PAD = [512, 512, 384, 256, 128]


def _vector_mesh():
    return plsc.VectorSubcoreMesh(core_axis_name="c", subcore_axis_name="s",
                                  num_cores=NCORE, num_subcores=NSUB)


def _sc_deg(dst_m, ones, zeros):
    rows_half = ROWS_E // NCORE
    rows_sub = rows_half // NSUB

    @functools.partial(
        pl.kernel,
        out_type=jax.ShapeDtypeStruct((NCORE, NPAD, 128), F32),
        mesh=_vector_mesh(),
        scratch_types=[
            pltpu.VMEM((rows_sub, 128), jnp.int32),
            pltpu.VMEM((128, 128), F32),
            pltpu.VMEM_SHARED((NPAD, 128), F32),
        ],
    )
    def k(dst_hbm, ones_hbm, zeros_hbm, out_hbm, didx, ones_v, table):
        core = lax.axis_index("c")
        sub = lax.axis_index("s")
        pltpu.sync_copy(
            dst_hbm.at[pl.ds(core * rows_half + sub * rows_sub, rows_sub)], didx)
        pltpu.sync_copy(ones_hbm, ones_v)
        nbase = sub * ROWS_SUB_N
        pltpu.sync_copy(zeros_hbm.at[pl.ds(nbase, ROWS_SUB_N)],
                        table.at[pl.ds(nbase, ROWS_SUB_N)])
        plsc.subcore_barrier()

        @pl.loop(0, rows_sub)
        def _(t):
            pltpu.sync_copy(ones_v, table.at[didx.at[t]], add=True)

        plsc.subcore_barrier()
        pltpu.sync_copy(table.at[pl.ds(nbase, ROWS_SUB_N)],
                        out_hbm.at[core].at[pl.ds(nbase, ROWS_SUB_N)])

    return k(dst_m, ones, zeros)


def _make_sc_edge(nc, chunk_split):
    rows_sub = 40
    phases = 2 if chunk_split else 1
    ncpc = nc // NCORE
    out_shape = ((nc, NPAD, WC) if chunk_split else (NCORE, nc, NPAD, WC))

    @functools.partial(
        pl.kernel,
        out_type=jax.ShapeDtypeStruct(out_shape, F32),
        mesh=_vector_mesh(),
        scratch_types=[
            pltpu.VMEM((rows_sub, 128), jnp.int32),
            pltpu.VMEM((rows_sub, 128), jnp.int32),
            pltpu.VMEM((128, WC), F32),
            pltpu.VMEM((128, WC), F32),
            pltpu.VMEM_SHARED((NPAD, WC), F32),
            pltpu.SemaphoreType.DMA,
            pltpu.SemaphoreType.DMA,
            pltpu.SemaphoreType.DMA,
            pltpu.SemaphoreType.DMA,
        ],
    )
    def k(y_hbm, src_hbm, dst_hbm, zeros_hbm, out_hbm, sidx, didx, rows0,
          rows1, table, gsem0, gsem1, ssem0, ssem1):
        core = lax.axis_index("c")
        sub = lax.axis_index("s")
        nbase = sub * ROWS_SUB_N

        def edge_pipeline(c):
            def _gather(t, buf, sem):
                pltpu.async_copy(y_hbm.at[c].at[sidx.at[t]], buf, sem)

            def _gather_wait(t, buf, sem):
                pltpu.make_async_copy(y_hbm.at[c].at[sidx.at[t]], buf,
                                      sem).wait()

            def _scatter(t, buf, sem):
                pltpu.async_copy(buf, table.at[didx.at[t]], sem, add=True)

            def _scatter_wait(t, buf, sem):
                pltpu.make_async_copy(buf, table.at[didx.at[t]], sem).wait()

            _gather(0, rows0, gsem0)
            _gather_wait(0, rows0, gsem0)
            _scatter(0, rows0, ssem0)
            _gather(1, rows1, gsem1)

            @pl.loop(1, rows_sub - 1, step=2)
            def _(t):
                _gather_wait(t, rows1, gsem1)
                _scatter(t, rows1, ssem1)
                _scatter_wait(t - 1, rows0, ssem0)
                _gather(t + 1, rows0, gsem0)
                _gather_wait(t + 1, rows0, gsem0)
                _scatter(t + 1, rows0, ssem0)
                _scatter_wait(t, rows1, ssem1)
                _gather(t + 2, rows1, gsem1)

            _gather_wait(rows_sub - 1, rows1, gsem1)
            _scatter(rows_sub - 1, rows1, ssem1)
            _scatter_wait(rows_sub - 2, rows0, ssem0)
            _scatter_wait(rows_sub - 1, rows1, ssem1)

        def load_idx(ibase):
            pltpu.sync_copy(src_hbm.at[pl.ds(ibase, rows_sub)], sidx)
            pltpu.sync_copy(dst_hbm.at[pl.ds(ibase, rows_sub)], didx)

        if chunk_split:
            for j in range(ncpc):
                c = core * ncpc + j
                pltpu.sync_copy(y_hbm.at[c].at[pl.ds(nbase, ROWS_SUB_N)],
                                table.at[pl.ds(nbase, ROWS_SUB_N)])
                plsc.subcore_barrier()
                for p in range(phases):
                    load_idx(sub * (phases * rows_sub) + p * rows_sub)
                    edge_pipeline(c)
                plsc.subcore_barrier()
                pltpu.sync_copy(table.at[pl.ds(nbase, ROWS_SUB_N)],
                                out_hbm.at[c].at[pl.ds(nbase, ROWS_SUB_N)])
        else:
            load_idx(core * (ROWS_E // NCORE) + sub * rows_sub)
            for c in range(nc):
                @pl.when(core == 0)
                def _():
                    pltpu.sync_copy(y_hbm.at[c].at[pl.ds(nbase, ROWS_SUB_N)],
                                    table.at[pl.ds(nbase, ROWS_SUB_N)])

                @pl.when(core == 1)
                def _():
                    pltpu.sync_copy(zeros_hbm.at[pl.ds(nbase, ROWS_SUB_N)],
                                    table.at[pl.ds(nbase, ROWS_SUB_N)])

                plsc.subcore_barrier()
                edge_pipeline(c)
                plsc.subcore_barrier()
                pltpu.sync_copy(
                    table.at[pl.ds(nbase, ROWS_SUB_N)],
                    out_hbm.at[core].at[c].at[pl.ds(nbase, ROWS_SUB_N)])

    return k


_NB = 1024
_G = NPAD // _NB


def _tc_matmul1(x_p, w1_p):
    def body(x_ref, w_ref, xw_ref):
        xw = jnp.dot(x_ref[...], w_ref[...], preferred_element_type=F32)
        for c in range(4):
            xw_ref[c] = xw[:, c * WC:(c + 1) * WC]

    return pl.pallas_call(
        body,
        grid=(_G,),
        in_specs=[
            pl.BlockSpec((_NB, 32), lambda i: (i, 0)),
            pl.BlockSpec((32, 512), lambda i: (0, 0)),
        ],
        out_specs=pl.BlockSpec((4, _NB, WC), lambda i: (0, i, 0)),
        out_shape=jax.ShapeDtypeStruct((4, NPAD, WC), F32),
    )(x_p, w1_p)


def _tc_dinv_scale(xw, deg):
    def body(xw_ref, deg_ref, y_ref, dinv_ref):
        d = deg_ref[0, :, 0:1] + deg_ref[1, :, 0:1] + 1.0
        dinv = lax.rsqrt(d)
        dinv_ref[...] = dinv
        for c in range(4):
            y_ref[c] = xw_ref[c] * dinv

    return pl.pallas_call(
        body,
        grid=(_G,),
        in_specs=[
            pl.BlockSpec((4, _NB, WC), lambda i: (0, i, 0)),
            pl.BlockSpec((2, _NB, 128), lambda i: (0, i, 0)),
        ],
        out_specs=[
            pl.BlockSpec((4, _NB, WC), lambda i: (0, i, 0)),
            pl.BlockSpec((_NB, 1), lambda i: (i, 0)),
        ],
        out_shape=[
            jax.ShapeDtypeStruct((4, NPAD, WC), F32),
            jax.ShapeDtypeStruct((NPAD, 1), F32),
        ],
    )(xw, deg)


def _tc_mid(s, dinv, b_r, w_r, nc_out):
    partial = s.ndim == 4
    nc_in = s.shape[1] if partial else s.shape[0]
    d_out = nc_out * WC

    def body(s_ref, dinv_ref, b_ref, w_ref, o_ref):
        dinv = dinv_ref[...]
        acc = jnp.zeros((_NB, d_out), F32)
        for c in range(nc_in):
            sc = (s_ref[0, c] + s_ref[1, c]) if partial else s_ref[c]
            h = jnp.maximum(sc * dinv + b_ref[c], 0.0)
            acc = acc + jnp.dot(h, w_ref[c], preferred_element_type=F32)
        y = acc * dinv
        for c2 in range(nc_out):
            o_ref[c2] = y[:, c2 * WC:(c2 + 1) * WC]

    s_spec = (pl.BlockSpec((2, nc_in, _NB, WC), lambda i: (0, 0, i, 0))
              if partial else
              pl.BlockSpec((nc_in, _NB, WC), lambda i: (0, i, 0)))
    return pl.pallas_call(
        body,
        grid=(_G,),
        in_specs=[
            s_spec,
            pl.BlockSpec((_NB, 1), lambda i: (i, 0)),
            pl.BlockSpec((nc_in, 1, WC), lambda i: (0, 0, 0)),
            pl.BlockSpec((nc_in, WC, d_out), lambda i: (0, 0, 0)),
        ],
        out_specs=pl.BlockSpec((nc_out, _NB, WC), lambda i: (0, i, 0)),
        out_shape=jax.ShapeDtypeStruct((nc_out, NPAD, WC), F32),
    )(s, dinv, b_r, w_r)


def _tc_final(s5, dinv, b_r, batch_p, wlin_p, blin_p):
    def body(s_ref, dinv_ref, b_ref, batch_ref, w_ref, blin_ref, o_ref, pooled):
        i = pl.program_id(0)

        @pl.when(i == 0)
        def _():
            pooled[...] = jnp.full((32, WC), -jnp.inf, F32)

        dinv = dinv_ref[...]
        h = jnp.maximum((s_ref[0, 0] + s_ref[1, 0]) * dinv + b_ref[0], 0.0)
        bvec = batch_ref[...]
        glo = bvec[0, 0]
        ghi = jnp.minimum(bvec[_NB - 1, 0], 31)

        def upd(g, carry):
            m = bvec == g
            vals = jnp.max(jnp.where(m, h, -jnp.inf), axis=0, keepdims=True)
            pooled[pl.ds(g, 1), :] = jnp.maximum(pooled[pl.ds(g, 1), :], vals)
            return carry

        lax.fori_loop(glo, ghi + 1, upd, 0)

        @pl.when(i == _G - 1)
        def _():
            p = pooled[...]
            z = jnp.dot(p, w_ref[...], preferred_element_type=F32)
            o_ref[...] = jax.nn.sigmoid(z + blin_ref[...])

    return pl.pallas_call(
        body,
        grid=(_G,),
        in_specs=[
            pl.BlockSpec((2, 1, _NB, WC), lambda i: (0, 0, i, 0)),
            pl.BlockSpec((_NB, 1), lambda i: (i, 0)),
            pl.BlockSpec((1, 1, WC), lambda i: (0, 0, 0)),
            pl.BlockSpec((_NB, 1), lambda i: (i, 0)),
            pl.BlockSpec((WC, 1), lambda i: (0, 0)),
            pl.BlockSpec((1, 1), lambda i: (0, 0)),
        ],
        out_specs=pl.BlockSpec((32, 1), lambda i: (0, 0)),
        out_shape=jax.ShapeDtypeStruct((32, 1), F32),
        scratch_shapes=[pltpu.VMEM((32, WC), F32)],
    )(s5, dinv, b_r, batch_p, wlin_p, blin_p)


def _pad2(w, r, c):
    return jnp.zeros((r, c), F32).at[:w.shape[0], :w.shape[1]].set(w)


def kernel(x, edge_index, batch, W1, b1, W2, b2, W3, b3, W4, b4, W5, b5,
           W_lin, b_lin):
    sink = (jnp.arange(EPAD - E, dtype=jnp.int32) % (NPAD - N)) + N
    src_m = jnp.concatenate([edge_index[0], sink]).reshape(ROWS_E, 128)
    dst_m = jnp.concatenate([edge_index[1], sink]).reshape(ROWS_E, 128)
    x_p = _pad2(x, NPAD, 32)
    w1_p = _pad2(W1, 32, 512)
    ws = [W2, W3, W4, W5]
    w_rs = []
    for li in range(4):
        w_rs.append(_pad2(ws[li], DIMS_PAD[li], DIMS_PAD[li + 1])
                    .reshape(LAYER_NC[li], WC, DIMS_PAD[li + 1]))
    b_rs = []
    for li, b in enumerate([b1, b2, b3, b4, b5]):
        b_rs.append(jnp.zeros((DIMS_PAD[li],), F32).at[:b.shape[0]].set(b)
                    .reshape(LAYER_NC[li], 1, WC))
    batch_p = jnp.concatenate(
        [batch, jnp.full((NPAD - N,), 32, jnp.int32)]).reshape(NPAD, 1)
    wlin_p = _pad2(W_lin, WC, 1)
    blin_p = b_lin.reshape(1, 1)
    ones = jnp.ones((128, 128), F32)
    zeros = jnp.zeros((NPAD, 128), F32)

    deg = _sc_deg(dst_m, ones, zeros)
    xw = _tc_matmul1(x_p, w1_p)
    y, dinv = _tc_dinv_scale(xw, deg)
    chunk_split = [True, True, False, True, False]
    for li in range(5):
        s = _make_sc_edge(LAYER_NC[li], chunk_split[li])(y, src_m, dst_m,
                                                         zeros)
        if li < 4:
            y = _tc_mid(s, dinv, b_rs[li], w_rs[li], LAYER_NC[li + 1])
    return _tc_final(s, dinv, b_rs[4], batch_p, wlin_p, blin_p)


# speedup vs baseline: 1.2079x; 1.0049x over previous
import functools

import jax
import jax.numpy as jnp
from jax import lax
from jax.experimental import pallas as pl
from jax.experimental.pallas import tpu as pltpu
from jax.experimental.pallas import tpu_sc as plsc

N = 10000
E = 160000
NPAD = 10240
EPAD = 163840
ROWS_E = EPAD // 128
NSUB = 16
NCORE = 2
ROWS_SUB_N = NPAD // NSUB
WC = 128
F32 = jnp.float32

LAYER_NC = [4, 4, 3, 2, 1]
DIMS_PAD = [512, 512, 384, 256, 128]


def _vector_mesh():
    return plsc.VectorSubcoreMesh(core_axis_name="c", subcore_axis_name="s",
                                  num_cores=NCORE, num_subcores=NSUB)


def _sc_deg(dst_m, ones, zeros):
    rows_half = ROWS_E // NCORE
    rows_sub = rows_half // NSUB

    @functools.partial(
        pl.kernel,
        out_type=jax.ShapeDtypeStruct((NCORE, NPAD, 128), F32),
        mesh=_vector_mesh(),
        scratch_types=[
            pltpu.VMEM((rows_sub, 128), jnp.int32),
            pltpu.VMEM((128, 128), F32),
            pltpu.VMEM_SHARED((NPAD, 128), F32),
        ],
    )
    def k(dst_hbm, ones_hbm, zeros_hbm, out_hbm, didx, ones_v, table):
        core = lax.axis_index("c")
        sub = lax.axis_index("s")
        pltpu.sync_copy(
            dst_hbm.at[pl.ds(core * rows_half + sub * rows_sub, rows_sub)], didx)
        pltpu.sync_copy(ones_hbm, ones_v)
        nbase = sub * ROWS_SUB_N
        pltpu.sync_copy(zeros_hbm.at[pl.ds(nbase, ROWS_SUB_N)],
                        table.at[pl.ds(nbase, ROWS_SUB_N)])
        plsc.subcore_barrier()

        @pl.loop(0, rows_sub)
        def _(t):
            pltpu.sync_copy(ones_v, table.at[didx.at[t]], add=True)

        plsc.subcore_barrier()
        pltpu.sync_copy(table.at[pl.ds(nbase, ROWS_SUB_N)],
                        out_hbm.at[core].at[pl.ds(nbase, ROWS_SUB_N)])

    return k(dst_m, ones, zeros)


def _make_sc_edge(nc, chunk_split):
    rows_sub = 40
    phases = 2 if chunk_split else 1
    ncpc = nc // NCORE
    out_shape = ((nc, NPAD, WC) if chunk_split else (NCORE, nc, NPAD, WC))

    @functools.partial(
        pl.kernel,
        out_type=jax.ShapeDtypeStruct(out_shape, F32),
        mesh=_vector_mesh(),
        scratch_types=[
            pltpu.VMEM((rows_sub, 128), jnp.int32),
            pltpu.VMEM((rows_sub, 128), jnp.int32),
            pltpu.VMEM((128, WC), F32),
            pltpu.VMEM((128, WC), F32),
            pltpu.VMEM_SHARED((NPAD, WC), F32),
            pltpu.SemaphoreType.DMA,
            pltpu.SemaphoreType.DMA,
            pltpu.SemaphoreType.DMA,
            pltpu.SemaphoreType.DMA,
        ],
    )
    def k(y_hbm, src_hbm, dst_hbm, zeros_hbm, out_hbm, sidx, didx, rows0,
          rows1, table, gsem0, gsem1, ssem0, ssem1):
        core = lax.axis_index("c")
        sub = lax.axis_index("s")
        nbase = sub * ROWS_SUB_N

        def edge_pipeline(c):
            def _gather(t, buf, sem):
                pltpu.async_copy(y_hbm.at[c].at[sidx.at[t]], buf, sem)

            def _gather_wait(t, buf, sem):
                pltpu.make_async_copy(y_hbm.at[c].at[sidx.at[t]], buf,
                                      sem).wait()

            def _scatter(t, buf, sem):
                pltpu.async_copy(buf, table.at[didx.at[t]], sem, add=True)

            def _scatter_wait(t, buf, sem):
                pltpu.make_async_copy(buf, table.at[didx.at[t]], sem).wait()

            _gather(0, rows0, gsem0)
            _gather_wait(0, rows0, gsem0)
            _scatter(0, rows0, ssem0)
            _gather(1, rows1, gsem1)

            @pl.loop(1, rows_sub - 1, step=2)
            def _(t):
                _gather_wait(t, rows1, gsem1)
                _scatter(t, rows1, ssem1)
                _scatter_wait(t - 1, rows0, ssem0)
                _gather(t + 1, rows0, gsem0)
                _gather_wait(t + 1, rows0, gsem0)
                _scatter(t + 1, rows0, ssem0)
                _scatter_wait(t, rows1, ssem1)
                _gather(t + 2, rows1, gsem1)

            _gather_wait(rows_sub - 1, rows1, gsem1)
            _scatter(rows_sub - 1, rows1, ssem1)
            _scatter_wait(rows_sub - 2, rows0, ssem0)
            _scatter_wait(rows_sub - 1, rows1, ssem1)

        def load_idx(ibase):
            pltpu.sync_copy(src_hbm.at[pl.ds(ibase, rows_sub)], sidx)
            pltpu.sync_copy(dst_hbm.at[pl.ds(ibase, rows_sub)], didx)

        if chunk_split:
            for j in range(ncpc):
                c = core * ncpc + j
                pltpu.sync_copy(y_hbm.at[c].at[pl.ds(nbase, ROWS_SUB_N)],
                                table.at[pl.ds(nbase, ROWS_SUB_N)])
                plsc.subcore_barrier()
                for p in range(phases):
                    load_idx(sub * (phases * rows_sub) + p * rows_sub)
                    edge_pipeline(c)
                plsc.subcore_barrier()
                pltpu.sync_copy(table.at[pl.ds(nbase, ROWS_SUB_N)],
                                out_hbm.at[c].at[pl.ds(nbase, ROWS_SUB_N)])
        else:
            load_idx(core * (ROWS_E // NCORE) + sub * rows_sub)
            for c in range(nc):
                @pl.when(core == 0)
                def _():
                    pltpu.sync_copy(y_hbm.at[c].at[pl.ds(nbase, ROWS_SUB_N)],
                                    table.at[pl.ds(nbase, ROWS_SUB_N)])

                @pl.when(core == 1)
                def _():
                    pltpu.sync_copy(zeros_hbm.at[pl.ds(nbase, ROWS_SUB_N)],
                                    table.at[pl.ds(nbase, ROWS_SUB_N)])

                plsc.subcore_barrier()
                edge_pipeline(c)
                plsc.subcore_barrier()
                pltpu.sync_copy(
                    table.at[pl.ds(nbase, ROWS_SUB_N)],
                    out_hbm.at[core].at[c].at[pl.ds(nbase, ROWS_SUB_N)])

    return k


_NB = 2048
_G = NPAD // _NB


def _tc_matmul1(x_p, w1_p):
    def body(x_ref, w_ref, xw_ref):
        xw = jnp.dot(x_ref[...], w_ref[...], preferred_element_type=F32)
        for c in range(4):
            xw_ref[c] = xw[:, c * WC:(c + 1) * WC]

    return pl.pallas_call(
        body,
        grid=(_G,),
        in_specs=[
            pl.BlockSpec((_NB, 32), lambda i: (i, 0)),
            pl.BlockSpec((32, 512), lambda i: (0, 0)),
        ],
        out_specs=pl.BlockSpec((4, _NB, WC), lambda i: (0, i, 0)),
        out_shape=jax.ShapeDtypeStruct((4, NPAD, WC), F32),
    )(x_p, w1_p)


def _tc_dinv_scale(xw, deg):
    def body(xw_ref, deg_ref, y_ref, dinv_ref):
        d = deg_ref[0, :, 0:1] + deg_ref[1, :, 0:1] + 1.0
        dinv = lax.rsqrt(d)
        dinv_ref[...] = dinv
        for c in range(4):
            y_ref[c] = xw_ref[c] * dinv

    return pl.pallas_call(
        body,
        grid=(_G,),
        in_specs=[
            pl.BlockSpec((4, _NB, WC), lambda i: (0, i, 0)),
            pl.BlockSpec((2, _NB, 128), lambda i: (0, i, 0)),
        ],
        out_specs=[
            pl.BlockSpec((4, _NB, WC), lambda i: (0, i, 0)),
            pl.BlockSpec((_NB, 1), lambda i: (i, 0)),
        ],
        out_shape=[
            jax.ShapeDtypeStruct((4, NPAD, WC), F32),
            jax.ShapeDtypeStruct((NPAD, 1), F32),
        ],
    )(xw, deg)


def _tc_mid(s, dinv, b_r, w_r, nc_out):
    partial = s.ndim == 4
    nc_in = s.shape[1] if partial else s.shape[0]
    d_out = nc_out * WC

    def body(s_ref, dinv_ref, b_ref, w_ref, o_ref):
        dinv = dinv_ref[...]
        acc = jnp.zeros((_NB, d_out), F32)
        for c in range(nc_in):
            sc = (s_ref[0, c] + s_ref[1, c]) if partial else s_ref[c]
            h = jnp.maximum(sc * dinv + b_ref[c], 0.0)
            acc = acc + jnp.dot(h, w_ref[c], preferred_element_type=F32)
        y = acc * dinv
        for c2 in range(nc_out):
            o_ref[c2] = y[:, c2 * WC:(c2 + 1) * WC]

    s_spec = (pl.BlockSpec((2, nc_in, _NB, WC), lambda i: (0, 0, i, 0))
              if partial else
              pl.BlockSpec((nc_in, _NB, WC), lambda i: (0, i, 0)))
    return pl.pallas_call(
        body,
        grid=(_G,),
        in_specs=[
            s_spec,
            pl.BlockSpec((_NB, 1), lambda i: (i, 0)),
            pl.BlockSpec((nc_in, 1, WC), lambda i: (0, 0, 0)),
            pl.BlockSpec((nc_in, WC, d_out), lambda i: (0, 0, 0)),
        ],
        out_specs=pl.BlockSpec((nc_out, _NB, WC), lambda i: (0, i, 0)),
        out_shape=jax.ShapeDtypeStruct((nc_out, NPAD, WC), F32),
    )(s, dinv, b_r, w_r)


def _tc_final(s5, dinv, b_r, batch_p, wlin_p, blin_p):
    def body(s_ref, dinv_ref, b_ref, batch_ref, w_ref, blin_ref, o_ref, pooled):
        i = pl.program_id(0)

        @pl.when(i == 0)
        def _():
            pooled[...] = jnp.full((32, WC), -jnp.inf, F32)

        dinv = dinv_ref[...]
        h = jnp.maximum((s_ref[0, 0] + s_ref[1, 0]) * dinv + b_ref[0], 0.0)
        bvec = batch_ref[...]
        glo = bvec[0, 0]
        ghi = jnp.minimum(bvec[_NB - 1, 0], 31)

        def upd(g, carry):
            m = bvec == g
            vals = jnp.max(jnp.where(m, h, -jnp.inf), axis=0, keepdims=True)
            pooled[pl.ds(g, 1), :] = jnp.maximum(pooled[pl.ds(g, 1), :], vals)
            return carry

        lax.fori_loop(glo, ghi + 1, upd, 0)

        @pl.when(i == _G - 1)
        def _():
            p = pooled[...]
            z = jnp.dot(p, w_ref[...], preferred_element_type=F32)
            o_ref[...] = jax.nn.sigmoid(z + blin_ref[...])

    return pl.pallas_call(
        body,
        grid=(_G,),
        in_specs=[
            pl.BlockSpec((2, 1, _NB, WC), lambda i: (0, 0, i, 0)),
            pl.BlockSpec((_NB, 1), lambda i: (i, 0)),
            pl.BlockSpec((1, 1, WC), lambda i: (0, 0, 0)),
            pl.BlockSpec((_NB, 1), lambda i: (i, 0)),
            pl.BlockSpec((WC, 1), lambda i: (0, 0)),
            pl.BlockSpec((1, 1), lambda i: (0, 0)),
        ],
        out_specs=pl.BlockSpec((32, 1), lambda i: (0, 0)),
        out_shape=jax.ShapeDtypeStruct((32, 1), F32),
        scratch_shapes=[pltpu.VMEM((32, WC), F32)],
    )(s5, dinv, b_r, batch_p, wlin_p, blin_p)


def _pad2(w, r, c):
    return jnp.zeros((r, c), F32).at[:w.shape[0], :w.shape[1]].set(w)


def kernel(x, edge_index, batch, W1, b1, W2, b2, W3, b3, W4, b4, W5, b5,
           W_lin, b_lin):
    sink = (jnp.arange(EPAD - E, dtype=jnp.int32) % (NPAD - N)) + N
    src_m = jnp.concatenate([edge_index[0], sink]).reshape(ROWS_E, 128)
    dst_m = jnp.concatenate([edge_index[1], sink]).reshape(ROWS_E, 128)
    x_p = _pad2(x, NPAD, 32)
    w1_p = _pad2(W1, 32, 512)
    ws = [W2, W3, W4, W5]
    w_rs = []
    for li in range(4):
        w_rs.append(_pad2(ws[li], DIMS_PAD[li], DIMS_PAD[li + 1])
                    .reshape(LAYER_NC[li], WC, DIMS_PAD[li + 1]))
    b_rs = []
    for li, b in enumerate([b1, b2, b3, b4, b5]):
        b_rs.append(jnp.zeros((DIMS_PAD[li],), F32).at[:b.shape[0]].set(b)
                    .reshape(LAYER_NC[li], 1, WC))
    batch_p = jnp.concatenate(
        [batch, jnp.full((NPAD - N,), 32, jnp.int32)]).reshape(NPAD, 1)
    wlin_p = _pad2(W_lin, WC, 1)
    blin_p = b_lin.reshape(1, 1)
    ones = jnp.ones((128, 128), F32)
    zeros = jnp.zeros((NPAD, 128), F32)

    deg = _sc_deg(dst_m, ones, zeros)
    xw = _tc_matmul1(x_p, w1_p)
    y, dinv = _tc_dinv_scale(xw, deg)
    chunk_split = [True, True, False, True, False]
    for li in range(5):
        s = _make_sc_edge(LAYER_NC[li], chunk_split[li])(y, src_m, dst_m,
                                                         zeros)
        if li < 4:
            y = _tc_mid(s, dinv, b_rs[li], w_rs[li], LAYER_NC[li + 1])
    return _tc_final(s, dinv, b_rs[4], batch_p, wlin_p, blin_p)
